# baseline (speedup 1.0000x reference)
import functools

import jax
import jax.numpy as jnp
from jax import lax
from jax.experimental import pallas as pl
from jax.experimental.pallas import tpu as pltpu
from jax.experimental.pallas import tpu_sc as plsc

B = 1024
V = 100000
TOP_K = 30
NORM_DIM = 16

NC = 2
NS = 16
NW = NC * NS
ROWS_PER_W = B // NW
CHUNK = 20000
NCHUNK = V // CHUNK
VREGS_PER_BLOCK = 50
BLOCKS_PER_CHUNK = CHUNK // (VREGS_PER_BLOCK * 16)

NEG_INF = float("-inf")
BIG_I32 = 1 << 30


def _lane():
    return lax.iota(jnp.int32, 16)


def _sortd(v):
    k, _ = plsc.sort_key_val(v, v, descending=True)
    return k


def _rev(v):
    return lax.rev(v, (0,))


def _extract(v, i):
    return jnp.max(jnp.where(_lane() == i, v, NEG_INF))


def _merge_top32(T0, T1, t30, v):
    c = jnp.where(v > t30, v, NEG_INF)
    s = _sortd(c)
    rs = _rev(s)
    hi = jnp.maximum(T0, rs)
    lo = jnp.minimum(T0, rs)
    T0n = _sortd(hi)
    rlo = _rev(_sortd(lo))
    T1n = _sortd(jnp.maximum(T1, rlo))
    return T0n, T1n, _extract(T1n, 13)


def _sc_topk_argmax():
    mesh = plsc.VectorSubcoreMesh(core_axis_name="c", subcore_axis_name="s")

    @functools.partial(
        pl.kernel,
        mesh=mesh,
        compiler_params=pltpu.CompilerParams(needs_layout_passes=False),
        out_type=[
            jax.ShapeDtypeStruct((B,), jnp.float32),
            jax.ShapeDtypeStruct((B,), jnp.int32),
        ],
        scratch_types=[
            pltpu.VMEM((CHUNK,), jnp.float32),
            pltpu.VMEM((ROWS_PER_W,), jnp.int32),
            pltpu.VMEM((32,), jnp.float32),
            pltpu.VMEM((16,), jnp.float32),
            pltpu.VMEM((16,), jnp.int32),
        ],
    )
    def body(logits_hbm, knn_hbm, labels_hbm, w2_hbm,
             out_p, out_t, buf, lab_v, w2_v, stage_p, stage_t):
        cid = lax.axis_index("c")
        sid = lax.axis_index("s")
        wid = sid * NC + cid
        base = wid * ROWS_PER_W
        lane = _lane()
        neg16 = jnp.full((16,), NEG_INF, jnp.float32)

        pltpu.sync_copy(labels_hbm.at[pl.ds(base, ROWS_PER_W)], lab_v)
        pltpu.sync_copy(w2_hbm, w2_v)
        wA = w2_v[pl.ds(0, 16)]
        wB = w2_v[pl.ds(16, 16)]

        def block_lane_max(vbase):
            accs = [neg16, neg16, neg16, neg16]
            for g in range(VREGS_PER_BLOCK):
                v = buf[pl.ds((vbase + g) * 16, 16)]
                accs[g % 4] = jnp.maximum(accs[g % 4], v)
            return jnp.maximum(jnp.maximum(accs[0], accs[1]),
                               jnp.maximum(accs[2], accs[3]))

        def scan_array(src_hbm, roff, want_topk):
            def chunk_body(ck, st):
                pltpu.sync_copy(src_hbm.at[pl.ds(roff + ck * CHUNK, CHUNK)],
                                buf)

                def blk_body(b, st2):
                    T0, T1, t30, gmax, gidx = st2
                    vbase = b * VREGS_PER_BLOCK
                    acc = block_lane_max(vbase)

                    if want_topk:
                        def do_topk(s3):
                            def g_body(g, s4):
                                v = buf[pl.ds((vbase + g) * 16, 16)]
                                return lax.cond(
                                    jnp.any(v > s4[2]),
                                    lambda s5: _merge_top32(*s5, v),
                                    lambda s5: s5,
                                    s4)
                            return lax.fori_loop(0, VREGS_PER_BLOCK,
                                                 g_body, s3)
                        T0, T1, t30 = lax.cond(
                            jnp.any(acc > t30), do_topk, lambda s3: s3,
                            (T0, T1, t30))

                    bm = jnp.max(acc)

                    def do_amax(st4):
                        def g_body(g, best):
                            v = buf[pl.ds((vbase + g) * 16, 16)]
                            cand = jnp.min(jnp.where(
                                v == bm, g * 16 + lane,
                                jnp.int32(BIG_I32)))
                            return jnp.minimum(best, cand)
                        pos = lax.fori_loop(0, VREGS_PER_BLOCK, g_body,
                                            jnp.int32(BIG_I32))
                        return bm, ck * CHUNK + vbase * 16 + pos

                    gmax, gidx = lax.cond(bm > gmax, do_amax,
                                          lambda st4: st4, (gmax, gidx))
                    return T0, T1, t30, gmax, gidx

                return lax.fori_loop(0, BLOCKS_PER_CHUNK, blk_body, st)

            init = (neg16, neg16, jnp.float32(NEG_INF),
                    jnp.float32(NEG_INF), jnp.int32(0))
            return lax.fori_loop(0, NCHUNK, chunk_body, init)

        def row_body(rl, carry):
            accP, accT = carry
            roff = (base + rl) * V

            T0, T1, _, _, gidx_lin = scan_array(logits_hbm, roff, True)
            _, _, _, _, gidx_knn = scan_array(knn_hbm, roff, False)

            T1m = jnp.where(lane < 14, T1, 0.0)
            partial = jnp.sum(T0 * wA) + jnp.sum(T1m * wB)

            lab_vec = lab_v[pl.ds((rl // 16) * 16, 16)]
            label_r = jnp.min(jnp.where(lane == rl % 16, lab_vec,
                                        jnp.int32(BIG_I32)))
            targ = jnp.where((gidx_lin != label_r) & (gidx_knn == label_r),
                             jnp.int32(1), jnp.int32(0))

            lsel = lane == (rl % 16)
            accP = jnp.where(lsel, partial, accP)
            accT = jnp.where(lsel, targ, accT)

            @pl.when(rl % 16 == 15)
            def _():
                stage_p[...] = accP
                stage_t[...] = accT
                off = base + (rl // 16) * 16
                pltpu.sync_copy(stage_p, out_p.at[pl.ds(off, 16)])
                pltpu.sync_copy(stage_t, out_t.at[pl.ds(off, 16)])

            return accP, accT

        lax.fori_loop(0, ROWS_PER_W, row_body,
                      (jnp.zeros((16,), jnp.float32),
                       jnp.zeros((16,), jnp.int32)))

    return body


_SC_KERNEL = _sc_topk_argmax()


def _tc_head(nf_ref, w1t_ref, b1_ref, w2f_ref, b2_ref, part_ref, out_ref):
    h = jnp.dot(nf_ref[...], w1t_ref[...],
                preferred_element_type=jnp.float32) + b1_ref[...]
    h = jnp.where(h >= 0, h, h * 0.1)
    out_ref[...] = (jnp.dot(h, w2f_ref[...],
                            preferred_element_type=jnp.float32)
                    + b2_ref[...] + part_ref[...])


def kernel(normalized_features, logits, knn_logits, labels, W1, b1, W2, b2):
    labels_i = labels.astype(jnp.int32)
    w2row = W2.reshape(-1)
    w2_top = jnp.concatenate(
        [w2row[:TOP_K], jnp.zeros((32 - TOP_K,), jnp.float32)])

    partial, targ = _SC_KERNEL(
        logits.reshape(-1), knn_logits.reshape(-1), labels_i, w2_top)

    out2d = pl.pallas_call(
        _tc_head,
        out_shape=jax.ShapeDtypeStruct((B, 1), jnp.float32),
    )(normalized_features, W1.T, b1.reshape(1, NORM_DIM),
      w2row[TOP_K:TOP_K + NORM_DIM].reshape(NORM_DIM, 1),
      b2.reshape(1, 1), partial.reshape(B, 1))

    return out2d.reshape(-1), targ.astype(jnp.bool_)
